# baseline (speedup 1.0000x reference)
import functools

import jax
import jax.numpy as jnp
from jax import lax
from jax.experimental import pallas as pl
from jax.experimental.pallas import tpu as pltpu
from jax.experimental.pallas import tpu_sc as plsc

N = 10000
E = 320000
D = 128

NC = 2
NS = 16
NW = NC * NS
EPW = E // NW
CHUNK = 80
NCHUNK = EPW // CHUNK
RPW = N // NS


def _scal_out(h, asrc, adst, scal_ref, gmax_ref):
    a_s = jnp.dot(h, asrc, preferred_element_type=jnp.float32)
    a_d = jnp.dot(h, adst, preferred_element_type=jnp.float32)
    gmax = jnp.max(a_s)
    t = gmax + a_d
    mub = jnp.maximum(t, 0.2 * t)
    scal_ref[:, 0:1] = a_s
    scal_ref[:, 1:2] = a_d
    scal_ref[:, 2:3] = mub
    gmax_ref[...] = jnp.full((1, 16), gmax, jnp.float32)


def _project_body(x_ref, w_ref, asrc_ref, adst_ref, h_ref, scal_ref,
                  gmax_ref):
    h = jnp.dot(x_ref[...], w_ref[...], preferred_element_type=jnp.float32)
    h_ref[...] = h
    _scal_out(h, asrc_ref[...], adst_ref[...], scal_ref, gmax_ref)


_project = pl.pallas_call(
    _project_body,
    out_shape=[
        jax.ShapeDtypeStruct((N, D), jnp.float32),
        jax.ShapeDtypeStruct((N, 8), jnp.float32),
        jax.ShapeDtypeStruct((1, 16), jnp.float32),
    ],
)


def _combine_x(pa, pb, dpack, h, scal, bias):
    a_s = scal[:, 0:1]
    a_d = scal[:, 1:2]
    mub = scal[:, 2:3]
    t = a_s + a_d
    e = jnp.maximum(t, 0.2 * t)
    wself = jnp.exp(e - mub)
    numer = pa + pb + wself * h
    den = dpack[:, 0:1] + dpack[:, 1:2] + wself + 1e-16
    return numer / den + bias


def _combine_body(pa_ref, pb_ref, dpack_ref, h_ref, scal_ref, bias_ref,
                  out_ref):
    out_ref[...] = _combine_x(pa_ref[...], pb_ref[...], dpack_ref[...],
                              h_ref[...], scal_ref[...], bias_ref[...])


_combine = pl.pallas_call(
    _combine_body,
    out_shape=jax.ShapeDtypeStruct((N, D), jnp.float32),
)


def _fused_body(pa_ref, pb_ref, dpack_ref, hp_ref, scalp_ref, bias_ref,
                w_ref, asrc_ref, adst_ref, h_ref, scal_ref, gmax_ref):
    x = _combine_x(pa_ref[...], pb_ref[...], dpack_ref[...], hp_ref[...],
                   scalp_ref[...], bias_ref[...])
    x = jnp.maximum(x, 0.0)
    h = jnp.dot(x, w_ref[...], preferred_element_type=jnp.float32)
    h_ref[...] = h
    _scal_out(h, asrc_ref[...], adst_ref[...], scal_ref, gmax_ref)


_fused = pl.pallas_call(
    _fused_body,
    out_shape=[
        jax.ShapeDtypeStruct((N, D), jnp.float32),
        jax.ShapeDtypeStruct((N, 8), jnp.float32),
        jax.ShapeDtypeStruct((1, 16), jnp.float32),
    ],
)


NB = 4


def _edge_body(pack_hbm, h_hbm, as_hbm, ad_hbm, gmax_hbm, z2d_hbm, z1d_hbm,
               numer_out, den_out,
               eidx, dstv, asg, adg, g_l, w_v, rows_bf, scaled,
               isem0, isem1, isem2, isem3, gsem0, gsem1, gsem2, gsem3,
               ssem0, ssem1, ssem2, ssem3, wsem0, wsem1, wsem2, wsem3,
               numer_sp, den_sp):
    c = lax.axis_index("c")
    s = lax.axis_index("s")
    wid = s * NC + c
    isem = (isem0, isem1, isem2, isem3)
    gsem = (gsem0, gsem1, gsem2, gsem3)
    ssem = (ssem0, ssem1, ssem2, ssem3)
    wsem = (wsem0, wsem1, wsem2, wsem3)

    pltpu.sync_copy(gmax_hbm, g_l)

    pltpu.sync_copy(z2d_hbm, numer_sp.at[pl.ds(s * RPW, RPW)])

    @pl.when(s == 0)
    def _():
        pltpu.sync_copy(z1d_hbm, den_sp)

    plsc.subcore_barrier()

    g16 = g_l[0, pl.ds(0, 16)]

    def idx_copy(ci, p):
        return pltpu.make_async_copy(pack_hbm.at[wid, ci], eidx.at[p],
                                     isem[p])

    def gather_descs(p):
        return (
            pltpu.make_async_copy(h_hbm.at[eidx.at[p, 0]], rows_bf.at[p],
                                  gsem[p]),
            pltpu.make_async_copy(as_hbm.at[eidx.at[p, 0]], asg.at[p],
                                  gsem[p]),
            pltpu.make_async_copy(ad_hbm.at[eidx.at[p, 1]], adg.at[p],
                                  gsem[p]),
        )

    def gather_start(p):
        for d in gather_descs(p):
            d.start()

    def gather_wait(p):
        for d in gather_descs(p):
            d.wait()

    def scatter_start(p):
        sp = p % 2
        pltpu.make_async_copy(scaled.at[sp], numer_sp.at[dstv.at[p]],
                              ssem[sp]).start(add=True)
        pltpu.make_async_copy(w_v.at[p], den_sp.at[dstv.at[p]],
                              wsem[sp]).start(add=True)

    def scatter_wait(p):
        sp = p % 2
        pltpu.make_async_copy(scaled.at[sp], numer_sp.at[dstv.at[p]],
                              ssem[sp]).wait()
        pltpu.make_async_copy(w_v.at[p], den_sp.at[dstv.at[p]],
                              wsem[sp]).wait()

    def compute_w(p):
        for v in range(CHUNK // 16):
            sl = pl.ds(v * 16, 16)
            dv = eidx[p, 1, sl]
            dstv[p, sl] = dv
            a = asg[p, sl]
            b = adg[p, sl]
            t = a + b
            e = jnp.maximum(t, 0.2 * t)
            t2 = g16 + b
            m = jnp.maximum(t2, 0.2 * t2)
            w_v[p, sl] = jnp.exp(e - m)

    def scale(p):
        sp = p % 2

        def scale_body(g, carry2):
            wv = w_v[p, pl.ds(g * 16, 16)]
            for k in range(16):
                wk = wv[k]
                j = g * 16 + k
                for q in range(D // 32):
                    x = rows_bf[p, j, pl.ds(q * 32, 32)]
                    lo, hi = plsc.unpack(x, format=plsc.PackFormat.INTERLEAVED)
                    scaled[sp, j, pl.ds(q * 32, 16)] = lo * wk
                    scaled[sp, j, pl.ds(q * 32 + 16, 16)] = hi * wk
            return carry2

        lax.fori_loop(0, CHUNK // 16, scale_body, 0)

    def step(ci, p, *, tail=False):
        r = (p + 2) % NB
        scatter_wait(r)

        if not tail:
            @pl.when(ci + 2 < NCHUNK)
            def _():
                idx_copy(ci + 2, r).wait()
                gather_start(r)

        gather_wait(p)
        compute_w(p)

        if not tail:
            @pl.when(ci + 4 < NCHUNK)
            def _():
                idx_copy(ci + 4, p).start()

        scale(p)
        scatter_start(p)

    idx_copy(0, 0).start()
    idx_copy(0, 0).wait()
    gather_start(0)
    idx_copy(1, 1).start()
    idx_copy(1, 1).wait()
    gather_start(1)
    idx_copy(2, 2).start()
    idx_copy(3, 3).start()

    idx_copy(2, 2).wait()
    gather_start(2)
    gather_wait(0)
    compute_w(0)
    idx_copy(4, 0).start()
    scale(0)
    scatter_start(0)

    idx_copy(3, 3).wait()
    gather_start(3)
    gather_wait(1)
    compute_w(1)
    idx_copy(5, 1).start()
    scale(1)
    scatter_start(1)

    scatter_wait(0)
    idx_copy(4, 0).wait()
    gather_start(0)
    gather_wait(2)
    compute_w(2)
    idx_copy(6, 2).start()
    scale(2)
    scatter_start(2)

    def quad_body(k, carry):
        step(4 * k + 3, 3)
        step(4 * k + 4, 0)
        step(4 * k + 5, 1)
        step(4 * k + 6, 2)
        return carry

    lax.fori_loop(0, (NCHUNK - 3) // 4, quad_body, 0)

    step(NCHUNK - 2, (NCHUNK - 2) % NB, tail=True)
    step(NCHUNK - 1, (NCHUNK - 1) % NB, tail=True)

    scatter_wait((NCHUNK - 2) % NB)
    scatter_wait((NCHUNK - 1) % NB)

    plsc.subcore_barrier()

    pltpu.sync_copy(numer_sp.at[pl.ds(s * RPW, RPW)],
                    numer_out.at[c, pl.ds(s * RPW, RPW)])

    @pl.when(s == 0)
    def _():
        pltpu.sync_copy(den_sp, den_out.at[c])


_edge_aggregate = pl.kernel(
    _edge_body,
    out_type=[
        jax.ShapeDtypeStruct((NC, N, D), jnp.float32),
        jax.ShapeDtypeStruct((NC, N), jnp.float32),
    ],
    mesh=plsc.VectorSubcoreMesh(core_axis_name="c", subcore_axis_name="s",
                                num_cores=NC, num_subcores=NS),
    compiler_params=pltpu.CompilerParams(use_tc_tiling_on_sc=False,
                                         needs_layout_passes=False),
    scratch_types=(
        [
            pltpu.VMEM((NB, 2, CHUNK), jnp.int32),
            pltpu.VMEM((NB, CHUNK), jnp.int32),
            pltpu.VMEM((NB, CHUNK), jnp.float32),
            pltpu.VMEM((NB, CHUNK), jnp.float32),
            pltpu.VMEM((1, 16), jnp.float32),
            pltpu.VMEM((NB, CHUNK), jnp.float32),
            pltpu.VMEM((NB, CHUNK, D), jnp.bfloat16),
            pltpu.VMEM((2, CHUNK, D), jnp.float32),
        ]
        + [pltpu.SemaphoreType.DMA] * (4 * NB)
        + [
            pltpu.VMEM_SHARED((N, D), jnp.float32),
            pltpu.VMEM_SHARED((N,), jnp.float32),
        ]
    ),
)


def kernel(z, edge_index, W, att_src, att_dst, bias):
    src3 = edge_index[0].reshape(NW, NCHUNK, CHUNK)
    dst3 = edge_index[1].reshape(NW, NCHUNK, CHUNK)
    pack = jnp.stack([src3, dst3], axis=2)
    z2d = jnp.zeros((RPW, D), jnp.float32)
    z1d = jnp.zeros((N,), jnp.float32)

    def swz(h):
        t = jnp.swapaxes(h.reshape(N, D // 32, 2, 16), 2, 3)
        return t.reshape(N, D).astype(jnp.bfloat16)

    def edge(h, scal, gmax16):
        return _edge_aggregate(
            pack, swz(h), scal[:, 0], scal[:, 1], gmax16, z2d, z1d)

    wgt = lambda li: (W[li], att_src[li].reshape(D, 1),
                      att_dst[li].reshape(D, 1))

    h, scal, g16 = _project(z, *wgt(0))
    numer, den = edge(h, scal, g16)
    h, scal, g16 = _fused(numer[0], numer[1], den.T, h, scal,
                          bias[0].reshape(1, D), *wgt(1))
    numer, den = edge(h, scal, g16)
    h, scal, g16 = _fused(numer[0], numer[1], den.T, h, scal,
                          bias[1].reshape(1, D), *wgt(1))
    numer, den = edge(h, scal, g16)
    return _combine(numer[0], numer[1], den.T, h, scal,
                    bias[1].reshape(1, D))


# speedup vs baseline: 1.8316x; 1.8316x over previous
import functools

import jax
import jax.numpy as jnp
from jax import lax
from jax.experimental import pallas as pl
from jax.experimental.pallas import tpu as pltpu
from jax.experimental.pallas import tpu_sc as plsc

N = 10000
E = 320000
D = 128

NC = 2
NS = 16
NW = NC * NS
EPW = E // NW
CHUNK = 80
NCHUNK = EPW // CHUNK
RPW = N // NS


def _scal_out(h, asrc, adst, scal_ref, gmax_ref):
    a_s = jnp.dot(h, asrc, preferred_element_type=jnp.float32)
    a_d = jnp.dot(h, adst, preferred_element_type=jnp.float32)
    gmax = jnp.max(a_s)
    t = gmax + a_d
    mub = jnp.maximum(t, 0.2 * t)
    scal_ref[:, 0:1] = a_s
    scal_ref[:, 1:2] = a_d
    scal_ref[:, 2:3] = mub
    gmax_ref[...] = jnp.full((1, 16), gmax, jnp.float32)


def _project_body(x_ref, w_ref, asrc_ref, adst_ref, h_ref, scal_ref,
                  gmax_ref):
    h = jnp.dot(x_ref[...], w_ref[...], preferred_element_type=jnp.float32)
    h_ref[...] = h
    _scal_out(h, asrc_ref[...], adst_ref[...], scal_ref, gmax_ref)


_project = pl.pallas_call(
    _project_body,
    out_shape=[
        jax.ShapeDtypeStruct((N, D), jnp.float32),
        jax.ShapeDtypeStruct((N, 8), jnp.float32),
        jax.ShapeDtypeStruct((1, 16), jnp.float32),
    ],
)


def _combine_x(pa, pb, dpack, h, scal, bias):
    a_s = scal[:, 0:1]
    a_d = scal[:, 1:2]
    mub = scal[:, 2:3]
    t = a_s + a_d
    e = jnp.maximum(t, 0.2 * t)
    wself = jnp.exp(e - mub)
    numer = pa + pb + wself * h
    den = dpack[:, 0:1] + dpack[:, 1:2] + wself + 1e-16
    return numer / den + bias


def _combine_body(pa_ref, pb_ref, dpack_ref, h_ref, scal_ref, bias_ref,
                  out_ref):
    out_ref[...] = _combine_x(pa_ref[...], pb_ref[...], dpack_ref[...],
                              h_ref[...], scal_ref[...], bias_ref[...])


_combine = pl.pallas_call(
    _combine_body,
    out_shape=jax.ShapeDtypeStruct((N, D), jnp.float32),
)


def _fused_body(pa_ref, pb_ref, dpack_ref, hp_ref, scalp_ref, bias_ref,
                w_ref, asrc_ref, adst_ref, h_ref, scal_ref, gmax_ref):
    x = _combine_x(pa_ref[...], pb_ref[...], dpack_ref[...], hp_ref[...],
                   scalp_ref[...], bias_ref[...])
    x = jnp.maximum(x, 0.0)
    h = jnp.dot(x, w_ref[...], preferred_element_type=jnp.float32)
    h_ref[...] = h
    _scal_out(h, asrc_ref[...], adst_ref[...], scal_ref, gmax_ref)


_fused = pl.pallas_call(
    _fused_body,
    out_shape=[
        jax.ShapeDtypeStruct((N, D), jnp.float32),
        jax.ShapeDtypeStruct((N, 8), jnp.float32),
        jax.ShapeDtypeStruct((1, 16), jnp.float32),
    ],
)


NB = 4


def _edge_body(pack_hbm, h_hbm, as_hbm, ad_hbm, gmax_hbm, z2d_hbm, z1d_hbm,
               numer_out, den_out,
               eidx, dstv, asg, adg, g_l, w_v, rows_v,
               isem0, isem1, isem2, isem3, gsem0, gsem1, gsem2, gsem3,
               ssem0, ssem1, ssem2, ssem3, wsem0, wsem1, wsem2, wsem3,
               numer_sp, den_sp):
    c = lax.axis_index("c")
    s = lax.axis_index("s")
    wid = s * NC + c
    isem = (isem0, isem1, isem2, isem3)
    gsem = (gsem0, gsem1, gsem2, gsem3)
    ssem = (ssem0, ssem1, ssem2, ssem3)
    wsem = (wsem0, wsem1, wsem2, wsem3)

    pltpu.sync_copy(gmax_hbm, g_l)

    pltpu.sync_copy(z2d_hbm, numer_sp.at[pl.ds(s * RPW, RPW)])

    @pl.when(s == 0)
    def _():
        pltpu.sync_copy(z1d_hbm, den_sp)

    plsc.subcore_barrier()

    g16 = g_l[0, pl.ds(0, 16)]

    def idx_copy(ci, p):
        return pltpu.make_async_copy(pack_hbm.at[wid, ci], eidx.at[p],
                                     isem[p])

    def gather_descs(p):
        return (
            pltpu.make_async_copy(h_hbm.at[eidx.at[p, 0]], rows_v.at[p],
                                  gsem[p]),
            pltpu.make_async_copy(as_hbm.at[eidx.at[p, 0]], asg.at[p],
                                  gsem[p]),
            pltpu.make_async_copy(ad_hbm.at[eidx.at[p, 1]], adg.at[p],
                                  gsem[p]),
        )

    def gather_start(p):
        for d in gather_descs(p):
            d.start()

    def gather_wait(p):
        for d in gather_descs(p):
            d.wait()

    def scatter_start(p):
        pltpu.make_async_copy(rows_v.at[p], numer_sp.at[dstv.at[p]],
                              ssem[p]).start(add=True)
        pltpu.make_async_copy(w_v.at[p], den_sp.at[dstv.at[p]],
                              wsem[p]).start(add=True)

    def scatter_wait(p):
        pltpu.make_async_copy(rows_v.at[p], numer_sp.at[dstv.at[p]],
                              ssem[p]).wait()
        pltpu.make_async_copy(w_v.at[p], den_sp.at[dstv.at[p]],
                              wsem[p]).wait()

    def compute_w(p):
        for v in range(CHUNK // 16):
            sl = pl.ds(v * 16, 16)
            dv = eidx[p, 1, sl]
            dstv[p, sl] = dv
            a = asg[p, sl]
            b = adg[p, sl]
            t = a + b
            e = jnp.maximum(t, 0.2 * t)
            t2 = g16 + b
            m = jnp.maximum(t2, 0.2 * t2)
            w_v[p, sl] = jnp.exp(e - m)

    def scale(p):
        def scale_body(g, carry2):
            wv = w_v[p, pl.ds(g * 16, 16)]
            for k in range(16):
                wk = wv[k]
                j = g * 16 + k
                for q in range(D // 16):
                    qs = pl.ds(q * 16, 16)
                    rows_v[p, j, qs] = rows_v[p, j, qs] * wk
            return carry2

        lax.fori_loop(0, CHUNK // 16, scale_body, 0)

    def step(ci, p, *, tail=False):
        r = (p + 2) % NB

        if not tail:
            @pl.when(ci + 2 < NCHUNK)
            def _():
                scatter_wait(r)
                idx_copy(ci + 2, r).wait()
                gather_start(r)

        gather_wait(p)
        compute_w(p)

        if not tail:
            @pl.when(ci + 4 < NCHUNK)
            def _():
                idx_copy(ci + 4, p).start()

        scale(p)
        scatter_start(p)

    idx_copy(0, 0).start()
    idx_copy(0, 0).wait()
    gather_start(0)
    idx_copy(1, 1).start()
    idx_copy(1, 1).wait()
    gather_start(1)
    idx_copy(2, 2).start()
    idx_copy(3, 3).start()

    idx_copy(2, 2).wait()
    gather_start(2)
    gather_wait(0)
    compute_w(0)
    idx_copy(4, 0).start()
    scale(0)
    scatter_start(0)

    idx_copy(3, 3).wait()
    gather_start(3)
    gather_wait(1)
    compute_w(1)
    idx_copy(5, 1).start()
    scale(1)
    scatter_start(1)

    scatter_wait(0)
    idx_copy(4, 0).wait()
    gather_start(0)
    gather_wait(2)
    compute_w(2)
    idx_copy(6, 2).start()
    scale(2)
    scatter_start(2)

    def quad_body(k, carry):
        step(4 * k + 3, 3)
        step(4 * k + 4, 0)
        step(4 * k + 5, 1)
        step(4 * k + 6, 2)
        return carry

    lax.fori_loop(0, (NCHUNK - 3) // 4, quad_body, 0)

    step(NCHUNK - 2, (NCHUNK - 2) % NB, tail=True)
    step(NCHUNK - 1, (NCHUNK - 1) % NB, tail=True)

    scatter_wait((NCHUNK - 4) % NB)
    scatter_wait((NCHUNK - 3) % NB)
    scatter_wait((NCHUNK - 2) % NB)
    scatter_wait((NCHUNK - 1) % NB)

    plsc.subcore_barrier()

    pltpu.sync_copy(numer_sp.at[pl.ds(s * RPW, RPW)],
                    numer_out.at[c, pl.ds(s * RPW, RPW)])

    @pl.when(s == 0)
    def _():
        pltpu.sync_copy(den_sp, den_out.at[c])


_edge_aggregate = pl.kernel(
    _edge_body,
    out_type=[
        jax.ShapeDtypeStruct((NC, N, D), jnp.float32),
        jax.ShapeDtypeStruct((NC, N), jnp.float32),
    ],
    mesh=plsc.VectorSubcoreMesh(core_axis_name="c", subcore_axis_name="s",
                                num_cores=NC, num_subcores=NS),
    compiler_params=pltpu.CompilerParams(use_tc_tiling_on_sc=False,
                                         needs_layout_passes=False),
    scratch_types=(
        [
            pltpu.VMEM((NB, 2, CHUNK), jnp.int32),
            pltpu.VMEM((NB, CHUNK), jnp.int32),
            pltpu.VMEM((NB, CHUNK), jnp.float32),
            pltpu.VMEM((NB, CHUNK), jnp.float32),
            pltpu.VMEM((1, 16), jnp.float32),
            pltpu.VMEM((NB, CHUNK), jnp.float32),
            pltpu.VMEM((NB, CHUNK, D), jnp.float32),
        ]
        + [pltpu.SemaphoreType.DMA] * (4 * NB)
        + [
            pltpu.VMEM_SHARED((N, D), jnp.float32),
            pltpu.VMEM_SHARED((N,), jnp.float32),
        ]
    ),
)


def kernel(z, edge_index, W, att_src, att_dst, bias):
    src3 = edge_index[0].reshape(NW, NCHUNK, CHUNK)
    dst3 = edge_index[1].reshape(NW, NCHUNK, CHUNK)
    pack = jnp.stack([src3, dst3], axis=2)
    z2d = jnp.zeros((RPW, D), jnp.float32)
    z1d = jnp.zeros((N,), jnp.float32)

    def edge(h, scal, gmax16):
        return _edge_aggregate(
            pack, h, scal[:, 0], scal[:, 1], gmax16, z2d, z1d)

    wgt = lambda li: (W[li], att_src[li].reshape(D, 1),
                      att_dst[li].reshape(D, 1))

    h, scal, g16 = _project(z, *wgt(0))
    numer, den = edge(h, scal, g16)
    h, scal, g16 = _fused(numer[0], numer[1], den.T, h, scal,
                          bias[0].reshape(1, D), *wgt(1))
    numer, den = edge(h, scal, g16)
    h, scal, g16 = _fused(numer[0], numer[1], den.T, h, scal,
                          bias[1].reshape(1, D), *wgt(1))
    numer, den = edge(h, scal, g16)
    return _combine(numer[0], numer[1], den.T, h, scal,
                    bias[1].reshape(1, D))
